# baseline (speedup 1.0000x reference)
import functools

import jax
import jax.numpy as jnp
from jax import lax
from jax.experimental import pallas as pl
from jax.experimental.pallas import tpu as pltpu
from jax.experimental.pallas import tpu_sc as plsc

IN_DIM = 1000000
DEPTH = 64
SEQ = 200
BATCH = 4096
ROWS = BATCH * SEQ
NC = 2
NS = 16
LANES = 16
NW = NC * NS
PER_W = ROWS // NW
CHUNK = 400
NCHUNK = PER_W // CHUNK
GSZ = 100
NG = CHUNK // GSZ
TOTAL_CHUNKS = ROWS // CHUNK


def _pos_encoding():
    half = DEPTH // 2
    positions = jnp.arange(SEQ, dtype=jnp.float32)[:, None]
    depths = jnp.arange(half, dtype=jnp.float32)[None, :] / half
    angle_rates = 1.0 / 10000.0 ** depths
    angle_rads = positions * angle_rates
    return jnp.concatenate([jnp.sin(angle_rads), jnp.cos(angle_rads)], axis=-1)


def _make_sc_kernel():
    mesh = plsc.VectorSubcoreMesh(core_axis_name="c", subcore_axis_name="s")

    @functools.partial(
        pl.kernel,
        mesh=mesh,
        compiler_params=pltpu.CompilerParams(use_tc_tiling_on_sc=False),
        out_type=jax.ShapeDtypeStruct((TOTAL_CHUNKS, CHUNK, DEPTH), jnp.float32),
        scratch_types=[
            pltpu.VMEM((NG, GSZ), jnp.int32),
            pltpu.VMEM((CHUNK, DEPTH), jnp.float32),
            pltpu.VMEM((CHUNK, DEPTH), jnp.float32),
            pltpu.SemaphoreType.DMA,
        ],
    )
    def k(idx_hbm, table_hbm, pos_hbm, out_hbm, idx_v, rows_v, pos_v, sem):
        wid = lax.axis_index("s") * NC + lax.axis_index("c")
        pltpu.sync_copy(pos_hbm, pos_v)

        def chunk_body(c, carry):
            cg = wid * NCHUNK + c
            pltpu.sync_copy(idx_hbm.at[cg], idx_v)
            for g in range(NG):
                pltpu.async_copy(
                    table_hbm.at[idx_v.at[g]],
                    rows_v.at[pl.ds(g * GSZ, GSZ)],
                    sem,
                ).wait()

            def row_body(i, rcarry):
                for j in range(DEPTH // LANES):
                    sl = pl.ds(j * LANES, LANES)
                    rows_v[i, sl] = rows_v[i, sl] + pos_v[i, sl]
                return rcarry

            lax.fori_loop(0, CHUNK, row_body, 0)
            pltpu.sync_copy(rows_v, out_hbm.at[cg])
            return carry

        lax.fori_loop(0, NCHUNK, chunk_body, 0)

    return k


def kernel(seq, table):
    idx = seq.astype(jnp.int32).reshape(TOTAL_CHUNKS, NG, GSZ)
    pos_tiled = jnp.tile(_pos_encoding(), (CHUNK // SEQ, 1))
    out = _make_sc_kernel()(idx, table, pos_tiled)
    return out.reshape(BATCH, SEQ, DEPTH)
